# baseline (speedup 1.0000x reference)
import functools

import jax
import jax.numpy as jnp
from jax import lax
from jax.experimental import pallas as pl
from jax.experimental.pallas import tpu as pltpu
from jax.experimental.pallas import tpu_sc as plsc

B, N, IN_CH, DIM, K, POS_H, MULT = 4, 2048, 128, 64, 16, 64, 4
H = DIM * MULT
T = 256
M = B * N * K
EPS = 1e-3
F32 = jnp.float32
NW = 32
CH = 128
NCHUNK = M // NW // CH


def _p1_body(x_ref, pos_ref, post_ref, wls_ref, bls_ref, wk_ref, bk_ref,
             wq_ref, bq_ref, wv_ref, bv_ref,
             keypos_ref, value_ref, query_ref, idx_ref):
    b = pl.program_id(0)
    t = pl.program_id(1)

    pos_t = post_ref[0]
    row0 = pl.multiple_of(t * T, T)
    pos_tile = pos_ref[0, pl.ds(row0, T), :]

    x = x_ref[0]
    h = jnp.dot(x, wls_ref[...], preferred_element_type=F32) + bls_ref[...]
    keyf = jnp.dot(h, wk_ref[...], preferred_element_type=F32) + bk_ref[...]
    keypos_ref[0] = jnp.concatenate(
        [keyf, pos_tile, jnp.zeros((T, IN_CH - DIM - 3), F32)], axis=1)
    value_ref[0] = jnp.dot(h, wq_ref[...], preferred_element_type=F32) + bq_ref[...]
    query_ref[0] = jnp.dot(h, wv_ref[...], preferred_element_type=F32) + bv_ref[...]

    sq_tile = jnp.sum(pos_tile * pos_tile, axis=1, keepdims=True)
    sq_full = jnp.sum(pos_t * pos_t, axis=0, keepdims=True)
    e = jnp.dot(pos_tile, pos_t, preferred_element_type=F32)
    dist = (sq_tile - 2.0 * e) + sq_full

    bits = dist.view(jnp.int32)
    bits = bits ^ ((bits >> 31) & jnp.int32(0x7FFFFFFF))
    iota = jax.lax.broadcasted_iota(jnp.int32, (T, N), 1)
    packed = (bits & jnp.int32(~2047)) | iota

    base = b * N
    idx_cols = []
    for k in range(K):
        mn = jnp.min(packed, axis=1, keepdims=True)
        hit = packed == mn
        packed = jnp.where(hit, jnp.int32(0x7FFFFFFF), packed)
        idx_cols.append((mn & 2047) + base)
    idx_ref[0] = jnp.concatenate(idx_cols, axis=1)


def _sc_gather_body(keytab, idx_hbm, oidx_hbm, keyg,
                    idx_v, oidx_v, rows_a, rows_b, sem_a, sem_b):
    wid = lax.axis_index("s") * 2 + lax.axis_index("c")
    pltpu.sync_copy(idx_hbm.at[wid], idx_v)
    pltpu.sync_copy(oidx_hbm.at[wid], oidx_v)

    def chunk(j, carry):
        a = pltpu.async_copy(keytab.at[idx_v.at[2 * j]], rows_a, sem_a)
        b2 = pltpu.async_copy(keytab.at[idx_v.at[2 * j + 1]], rows_b, sem_b)
        a.wait()
        c = pltpu.async_copy(rows_a, keyg.at[oidx_v.at[2 * j]], sem_a)
        b2.wait()
        d = pltpu.async_copy(rows_b, keyg.at[oidx_v.at[2 * j + 1]], sem_b)
        c.wait()
        d.wait()
        return carry

    lax.fori_loop(0, NCHUNK // 2, chunk, 0)


def _p2s_body(pos_ref, posg_ref, wp1_ref, bp1_ref, prel_ref, s1_ref):
    b = pl.program_id(0)
    t = pl.program_id(1)
    pos_tile = pos_ref[0]
    s_acc = jnp.zeros((1, POS_H), F32)
    q_acc = jnp.zeros((1, POS_H), F32)
    for k in range(K):
        pg = posg_ref[k, 0][:, DIM:DIM + 3]
        prel = pos_tile - pg
        prel_ref[k, 0] = prel
        p = jnp.dot(prel, wp1_ref[...], preferred_element_type=F32) + bp1_ref[...]
        s_acc = s_acc + jnp.sum(p, axis=0, keepdims=True)
        q_acc = q_acc + jnp.sum(p * p, axis=0, keepdims=True)

    @pl.when((b == 0) & (t == 0))
    def _():
        s1_ref[...] = jnp.zeros_like(s1_ref)

    s1_ref[...] += jnp.concatenate(
        [s_acc, q_acc, jnp.zeros((6, POS_H), F32)], axis=0)


def _p3_body(keyg_ref, query_ref, prel_ref, s1_ref,
             wp1_ref, bp1_ref, gp1_ref, bep1_ref, wp2_ref, bp2_ref,
             s2_ref):
    b = pl.program_id(0)
    t = pl.program_id(1)

    s1 = s1_ref[...]
    mean1 = s1[0:1, :] / M
    var1 = s1[1:2, :] / M - mean1 * mean1
    scale1 = gp1_ref[...] * jax.lax.rsqrt(var1 + EPS)
    shift1 = bep1_ref[...] - mean1 * scale1

    query = query_ref[0]

    s_acc = jnp.zeros((1, DIM), F32)
    g_acc = jnp.zeros((DIM, DIM), F32)
    for k in range(K):
        prel = prel_ref[k, 0]
        p = jnp.dot(prel, wp1_ref[...], preferred_element_type=F32) + bp1_ref[...]
        pe = jax.nn.relu(p * scale1 + shift1)
        pe = jnp.dot(pe, wp2_ref[...], preferred_element_type=F32) + bp2_ref[...]
        kg = keyg_ref[k, 0][:, :DIM]
        u = (query - kg) + pe
        s_acc = s_acc + jnp.sum(u, axis=0, keepdims=True)
        g_acc = g_acc + jax.lax.dot_general(
            u, u, (((0,), (0,)), ((), ())), preferred_element_type=F32)

    @pl.when((b == 0) & (t == 0))
    def _():
        s2_ref[...] = jnp.zeros_like(s2_ref)

    s2_ref[...] += jnp.concatenate(
        [s_acc, jnp.zeros((7, DIM), F32), g_acc], axis=0)


def _p4_body(keyg_ref, query_ref, value_ref, prel_ref, x_ref,
             s1_ref, s2_ref,
             wp1_ref, bp1_ref, gp1_ref, bep1_ref, wp2_ref, bp2_ref,
             wa1_ref, ba1_ref, ga1_ref, bea1_ref, wa2_ref, ba2_ref,
             wle_ref, ble_ref, y_ref):
    s1 = s1_ref[...]
    mean1 = s1[0:1, :] / M
    var1 = s1[1:2, :] / M - mean1 * mean1
    scale1 = gp1_ref[...] * jax.lax.rsqrt(var1 + EPS)
    shift1 = bep1_ref[...] - mean1 * scale1

    wa1 = wa1_ref[...]
    ba1 = ba1_ref[...]
    s_u = s2_ref[0:1, :]
    g_u = s2_ref[8:8 + DIM, :]
    sw = jnp.dot(s_u, wa1, preferred_element_type=F32) / M
    gw = jnp.dot(g_u, wa1, preferred_element_type=F32)
    e2 = jnp.sum(wa1 * gw, axis=0, keepdims=True) / M + 2.0 * ba1 * sw + ba1 * ba1
    mean2 = sw + ba1
    var2 = e2 - mean2 * mean2
    scale2 = ga1_ref[...] * jax.lax.rsqrt(var2 + EPS)
    shift2 = bea1_ref[...] - mean2 * scale2

    query = query_ref[0]
    value = value_ref[0]

    acc = jnp.zeros((T, DIM), F32)
    for k in range(K):
        prel = prel_ref[k, 0]
        p = jnp.dot(prel, wp1_ref[...], preferred_element_type=F32) + bp1_ref[...]
        pe = jax.nn.relu(p * scale1 + shift1)
        pe = jnp.dot(pe, wp2_ref[...], preferred_element_type=F32) + bp2_ref[...]
        kg = keyg_ref[k, 0][:, :DIM]
        u = (query - kg) + pe
        a = jnp.dot(u, wa1, preferred_element_type=F32) + ba1
        a = jax.nn.relu(a * scale2 + shift2)
        logit = jnp.dot(a, wa2_ref[...], preferred_element_type=F32) + ba2_ref[...]
        mx = jnp.max(logit, axis=1, keepdims=True)
        ex = jnp.exp(logit - mx)
        pr = ex / jnp.sum(ex, axis=1, keepdims=True)
        acc = acc + pr * (value + pe)
    y = jnp.dot(acc, wle_ref[...], preferred_element_type=F32) + ble_ref[...]
    y_ref[0] = y + x_ref[0]


def _full(shape):
    nd = len(shape)
    return pl.BlockSpec(shape, lambda b, t, _nd=nd: (0,) * _nd)


def _sc_gather(keypos_flat, idx3, oidx3):
    mesh = plsc.VectorSubcoreMesh(core_axis_name="c", subcore_axis_name="s")
    run = functools.partial(
        pl.kernel,
        out_type=jax.ShapeDtypeStruct((M, IN_CH), F32),
        mesh=mesh,
        scratch_types=[
            pltpu.VMEM((NCHUNK, CH), jnp.int32),
            pltpu.VMEM((NCHUNK, CH), jnp.int32),
            pltpu.VMEM((CH, IN_CH), F32),
            pltpu.VMEM((CH, IN_CH), F32),
            pltpu.SemaphoreType.DMA,
            pltpu.SemaphoreType.DMA,
        ],
    )(_sc_gather_body)
    return run(keypos_flat, idx3, oidx3)


def kernel(x, pos, W_ls, b_ls, W_k, b_k, W_q, b_q, W_v, b_v, W_p1, b_p1,
           g_p1, be_p1, W_p2, b_p2, W_a1, b_a1, g_a1, be_a1, W_a2, b_a2,
           W_le, b_le):
    pos_t = jnp.swapaxes(pos, 1, 2)
    r2 = lambda a: a.reshape(1, -1)
    grid = (B, N // T)

    p1 = pl.pallas_call(
        _p1_body,
        grid=grid,
        in_specs=[
            pl.BlockSpec((1, T, IN_CH), lambda b, t: (b, t, 0)),
            pl.BlockSpec((1, N, 3), lambda b, t: (b, 0, 0)),
            pl.BlockSpec((1, 3, N), lambda b, t: (b, 0, 0)),
            _full((IN_CH, DIM)), _full((1, DIM)),
            _full((DIM, DIM)), _full((1, DIM)),
            _full((DIM, DIM)), _full((1, DIM)),
            _full((DIM, DIM)), _full((1, DIM)),
        ],
        out_specs=[
            pl.BlockSpec((1, T, IN_CH), lambda b, t: (b, t, 0)),
            pl.BlockSpec((1, T, DIM), lambda b, t: (b, t, 0)),
            pl.BlockSpec((1, T, DIM), lambda b, t: (b, t, 0)),
            pl.BlockSpec((1, T, K), lambda b, t: (b, t, 0)),
        ],
        out_shape=[
            jax.ShapeDtypeStruct((B, N, IN_CH), F32),
            jax.ShapeDtypeStruct((B, N, DIM), F32),
            jax.ShapeDtypeStruct((B, N, DIM), F32),
            jax.ShapeDtypeStruct((B, N, K), jnp.int32),
        ],
    )
    keypos, value, query, idxg = p1(
        x, pos, pos_t, W_ls, r2(b_ls), W_k, r2(b_k), W_q, r2(b_q),
        W_v, r2(b_v))

    ar = jnp.arange(M, dtype=jnp.int32)
    oidx3 = ((ar % K) * (B * N) + ar // K).reshape(NW, NCHUNK, CH)
    idx3 = idxg.reshape(NW, NCHUNK, CH)
    keypos_flat = keypos.reshape(B * N, IN_CH)
    kpg_flat = _sc_gather(keypos_flat, idx3, oidx3)
    kpg = kpg_flat.reshape(K, B, N, IN_CH)

    p2s = pl.pallas_call(
        _p2s_body,
        grid=grid,
        in_specs=[
            pl.BlockSpec((1, T, 3), lambda b, t: (b, t, 0)),
            pl.BlockSpec((K, 1, T, IN_CH), lambda b, t: (0, b, t, 0)),
            _full((3, POS_H)), _full((1, POS_H)),
        ],
        out_specs=[
            pl.BlockSpec((K, 1, T, 3), lambda b, t: (0, b, t, 0)),
            pl.BlockSpec((8, POS_H), lambda b, t: (0, 0)),
        ],
        out_shape=[
            jax.ShapeDtypeStruct((K, B, N, 3), F32),
            jax.ShapeDtypeStruct((8, POS_H), F32),
        ],
    )
    prel, s1 = p2s(pos, kpg, W_p1, r2(b_p1))

    p3 = pl.pallas_call(
        _p3_body,
        grid=grid,
        in_specs=[
            pl.BlockSpec((K, 1, T, IN_CH), lambda b, t: (0, b, t, 0)),
            pl.BlockSpec((1, T, DIM), lambda b, t: (b, t, 0)),
            pl.BlockSpec((K, 1, T, 3), lambda b, t: (0, b, t, 0)),
            _full((8, POS_H)),
            _full((3, POS_H)), _full((1, POS_H)),
            _full((1, POS_H)), _full((1, POS_H)),
            _full((POS_H, DIM)), _full((1, DIM)),
        ],
        out_specs=pl.BlockSpec((8 + DIM, DIM), lambda b, t: (0, 0)),
        out_shape=jax.ShapeDtypeStruct((8 + DIM, DIM), F32),
    )
    s2 = p3(kpg, query, prel, s1,
            W_p1, r2(b_p1), r2(g_p1), r2(be_p1), W_p2, r2(b_p2))

    p4 = pl.pallas_call(
        _p4_body,
        grid=grid,
        in_specs=[
            pl.BlockSpec((K, 1, T, IN_CH), lambda b, t: (0, b, t, 0)),
            pl.BlockSpec((1, T, DIM), lambda b, t: (b, t, 0)),
            pl.BlockSpec((1, T, DIM), lambda b, t: (b, t, 0)),
            pl.BlockSpec((K, 1, T, 3), lambda b, t: (0, b, t, 0)),
            pl.BlockSpec((1, T, IN_CH), lambda b, t: (b, t, 0)),
            _full((8, POS_H)),
            _full((8 + DIM, DIM)),
            _full((3, POS_H)), _full((1, POS_H)),
            _full((1, POS_H)), _full((1, POS_H)),
            _full((POS_H, DIM)), _full((1, DIM)),
            _full((DIM, H)), _full((1, H)),
            _full((1, H)), _full((1, H)),
            _full((H, DIM)), _full((1, DIM)),
            _full((DIM, IN_CH)), _full((1, IN_CH)),
        ],
        out_specs=pl.BlockSpec((1, T, IN_CH), lambda b, t: (b, t, 0)),
        out_shape=jax.ShapeDtypeStruct((B, N, IN_CH), F32),
    )
    y = p4(kpg, query, value, prel, x, s1, s2,
           W_p1, r2(b_p1), r2(g_p1), r2(be_p1), W_p2, r2(b_p2),
           W_a1, r2(b_a1), r2(g_a1), r2(be_a1), W_a2, r2(b_a2),
           W_le, r2(b_le))
    return y


# speedup vs baseline: 1.2377x; 1.2377x over previous
import functools

import jax
import jax.numpy as jnp
from jax import lax
from jax.experimental import pallas as pl
from jax.experimental.pallas import tpu as pltpu
from jax.experimental.pallas import tpu_sc as plsc

B, N, IN_CH, DIM, K, POS_H, MULT = 4, 2048, 128, 64, 16, 64, 4
H = DIM * MULT
T = 256
M = B * N * K
EPS = 1e-3
F32 = jnp.float32
IMAX = 0x7FFFFFFF
NW = 32
CH = 128
NCHUNK = M // NW // CH


def _p1_body(x_ref, pos_ref, post_ref, wls_ref, bls_ref, wk_ref, bk_ref,
             wq_ref, bq_ref, wv_ref, bv_ref,
             keypos_ref, value_ref, query_ref, idx_ref, s1_ref):
    b = pl.program_id(0)
    t = pl.program_id(1)

    pos_t = post_ref[0]
    row0 = pl.multiple_of(t * T, T)
    pos_tile = pos_ref[0, pl.ds(row0, T), :]

    x = x_ref[0]
    h = jnp.dot(x, wls_ref[...], preferred_element_type=F32) + bls_ref[...]
    keyf = jnp.dot(h, wk_ref[...], preferred_element_type=F32) + bk_ref[...]
    keypos_ref[0] = jnp.concatenate(
        [keyf, pos_tile, jnp.zeros((T, IN_CH - DIM - 3), F32)], axis=1)
    value_ref[0] = jnp.dot(h, wq_ref[...], preferred_element_type=F32) + bq_ref[...]
    query_ref[0] = jnp.dot(h, wv_ref[...], preferred_element_type=F32) + bv_ref[...]

    sq_tile = jnp.sum(pos_tile * pos_tile, axis=1, keepdims=True)
    sq_full = jnp.sum(pos_t * pos_t, axis=0, keepdims=True)
    e = jnp.dot(pos_tile, pos_t, preferred_element_type=F32)
    dist = (sq_tile - 2.0 * e) + sq_full

    bits = dist.view(jnp.int32)
    bits = bits ^ ((bits >> 31) & jnp.int32(0x7FFFFFFF))
    iota = jax.lax.broadcasted_iota(jnp.int32, (T, N), 1)
    packed = (bits & jnp.int32(~2047)) | iota

    base = b * N
    idx_cols = []
    for k in range(K):
        mn = jnp.min(packed, axis=1, keepdims=True)
        hit = packed == mn
        packed = jnp.where(hit, jnp.int32(IMAX), packed)
        idx_cols.append((mn & 2047) + base)
    idx_ref[0] = jnp.concatenate(idx_cols, axis=1)

    mh = (packed == jnp.int32(IMAX)).astype(F32)
    pf = pos_ref[0, :, :]
    p0 = pf[:, 0:1]
    p1 = pf[:, 1:2]
    p2 = pf[:, 2:3]
    cat = jnp.concatenate(
        [pf, p0 * p0, p0 * p1, p0 * p2, p1 * p1, p1 * p2, p2 * p2], axis=1)
    sq12 = jnp.dot(mh, cat, preferred_element_type=F32)
    s = sq12[:, 0:3]
    q = sq12[:, 3:9]
    a0 = pos_tile[:, 0:1]
    a1 = pos_tile[:, 1:2]
    a2 = pos_tile[:, 2:3]
    s0 = s[:, 0:1]
    s1c = s[:, 1:2]
    s2c = s[:, 2:3]
    kf = jnp.float32(K)
    c_tile = jnp.concatenate([
        kf * pos_tile - s,
        kf * a0 * a0 - 2.0 * a0 * s0 + q[:, 0:1],
        kf * a0 * a1 - a0 * s1c - a1 * s0 + q[:, 1:2],
        kf * a0 * a2 - a0 * s2c - a2 * s0 + q[:, 2:3],
        kf * a1 * a1 - 2.0 * a1 * s1c + q[:, 3:4],
        kf * a1 * a2 - a1 * s2c - a2 * s1c + q[:, 4:5],
        kf * a2 * a2 - 2.0 * a2 * s2c + q[:, 5:6],
    ], axis=1)
    sums = jnp.sum(c_tile, axis=0, keepdims=True)
    upd = jnp.concatenate(
        [jnp.concatenate([sums, jnp.zeros((1, 128 - 9), F32)], axis=1),
         jnp.zeros((7, 128), F32)], axis=0)

    @pl.when((b == 0) & (t == 0))
    def _():
        s1_ref[...] = jnp.zeros_like(s1_ref)

    s1_ref[...] += upd


def _sc_gather_body(keytab, idx_hbm, oidx_hbm, keyg,
                    idx_v, oidx_v, rows_a, rows_b, sem_a, sem_b):
    wid = lax.axis_index("s") * 2 + lax.axis_index("c")
    pltpu.sync_copy(idx_hbm.at[wid], idx_v)
    pltpu.sync_copy(oidx_hbm.at[wid], oidx_v)

    def chunk(j, carry):
        a = pltpu.async_copy(keytab.at[idx_v.at[2 * j]], rows_a, sem_a)
        b2 = pltpu.async_copy(keytab.at[idx_v.at[2 * j + 1]], rows_b, sem_b)
        a.wait()
        c = pltpu.async_copy(rows_a, keyg.at[oidx_v.at[2 * j]], sem_a)
        b2.wait()
        d = pltpu.async_copy(rows_b, keyg.at[oidx_v.at[2 * j + 1]], sem_b)
        c.wait()
        d.wait()
        return carry

    lax.fori_loop(0, NCHUNK // 2, chunk, 0)


def _bn1_consts(s1_ref, wp1_ref, bp1_ref, gp1_ref, bep1_ref):
    sums = s1_ref[0:1, :]
    s3 = sums[:, 0:3] / M
    c6 = sums[:, 3:9] / M
    m0 = s3[:, 0:1]
    m1 = s3[:, 1:2]
    m2 = s3[:, 2:3]
    r0 = jnp.concatenate([c6[:, 0:1] - m0 * m0, c6[:, 1:2] - m0 * m1,
                          c6[:, 2:3] - m0 * m2], axis=1)
    r1 = jnp.concatenate([c6[:, 1:2] - m0 * m1, c6[:, 3:4] - m1 * m1,
                          c6[:, 4:5] - m1 * m2], axis=1)
    r2 = jnp.concatenate([c6[:, 2:3] - m0 * m2, c6[:, 4:5] - m1 * m2,
                          c6[:, 5:6] - m2 * m2], axis=1)
    cov = jnp.concatenate([r0, r1, r2], axis=0)
    wp1 = wp1_ref[...]
    cw = jnp.dot(cov, wp1, preferred_element_type=F32)
    var1 = jnp.sum(wp1 * cw, axis=0, keepdims=True)
    mean1 = jnp.dot(s3, wp1, preferred_element_type=F32) + bp1_ref[...]
    scale1 = gp1_ref[...] * jax.lax.rsqrt(var1 + EPS)
    shift1 = bep1_ref[...] - mean1 * scale1
    return scale1, shift1


def _p3_body(keyg_ref, pos_ref, query_ref, value_ref, s1_ref,
             wp1_ref, bp1_ref, gp1_ref, bep1_ref, wp2_ref, bp2_ref,
             wa1_ref, ba1_ref,
             u_ref, v_ref, s2_ref):
    b = pl.program_id(0)
    t = pl.program_id(1)

    scale1, shift1 = _bn1_consts(s1_ref, wp1_ref, bp1_ref, gp1_ref, bep1_ref)
    pos_tile = pos_ref[0]
    query = query_ref[0]
    value = value_ref[0]

    s_acc = jnp.zeros((1, H), F32)
    q_acc = jnp.zeros((1, H), F32)
    for k in range(K):
        kgrow = keyg_ref[k, 0]
        prel = pos_tile - kgrow[:, DIM:DIM + 3]
        p = jnp.dot(prel, wp1_ref[...], preferred_element_type=F32) + bp1_ref[...]
        pe = jax.nn.relu(p * scale1 + shift1)
        pe = jnp.dot(pe, wp2_ref[...], preferred_element_type=F32) + bp2_ref[...]
        u = (query - kgrow[:, :DIM]) + pe
        u_ref[k, 0] = u
        v_ref[k, 0] = value + pe
        a = jnp.dot(u, wa1_ref[...], preferred_element_type=F32) + ba1_ref[...]
        s_acc = s_acc + jnp.sum(a, axis=0, keepdims=True)
        q_acc = q_acc + jnp.sum(a * a, axis=0, keepdims=True)

    @pl.when((b == 0) & (t == 0))
    def _():
        s2_ref[...] = jnp.zeros_like(s2_ref)

    s2_ref[...] += jnp.concatenate(
        [s_acc, q_acc, jnp.zeros((6, H), F32)], axis=0)


def _p4_body(u_ref, v_ref, x_ref, s2_ref,
             wa1_ref, ba1_ref, ga1_ref, bea1_ref, wa2_ref, ba2_ref,
             wle_ref, ble_ref, y_ref):
    s2 = s2_ref[...]
    mean2 = s2[0:1, :] / M
    var2 = s2[1:2, :] / M - mean2 * mean2
    scale2 = ga1_ref[...] * jax.lax.rsqrt(var2 + EPS)
    shift2 = bea1_ref[...] - mean2 * scale2

    acc = jnp.zeros((T, DIM), F32)
    for k in range(K):
        u = u_ref[k, 0]
        a = jnp.dot(u, wa1_ref[...], preferred_element_type=F32) + ba1_ref[...]
        a = jax.nn.relu(a * scale2 + shift2)
        logit = jnp.dot(a, wa2_ref[...], preferred_element_type=F32) + ba2_ref[...]
        mx = jnp.max(logit, axis=1, keepdims=True)
        ex = jnp.exp(logit - mx)
        p = ex / jnp.sum(ex, axis=1, keepdims=True)
        acc = acc + p * v_ref[k, 0]
    y = jnp.dot(acc, wle_ref[...], preferred_element_type=F32) + ble_ref[...]
    y_ref[0] = y + x_ref[0]


def _full(shape):
    nd = len(shape)
    return pl.BlockSpec(shape, lambda b, t, _nd=nd: (0,) * _nd)


def _sc_gather(keypos_flat, idx3, oidx3):
    mesh = plsc.VectorSubcoreMesh(core_axis_name="c", subcore_axis_name="s")
    run = functools.partial(
        pl.kernel,
        out_type=jax.ShapeDtypeStruct((M, IN_CH), F32),
        mesh=mesh,
        scratch_types=[
            pltpu.VMEM((NCHUNK, CH), jnp.int32),
            pltpu.VMEM((NCHUNK, CH), jnp.int32),
            pltpu.VMEM((CH, IN_CH), F32),
            pltpu.VMEM((CH, IN_CH), F32),
            pltpu.SemaphoreType.DMA,
            pltpu.SemaphoreType.DMA,
        ],
    )(_sc_gather_body)
    return run(keypos_flat, idx3, oidx3)


def kernel(x, pos, W_ls, b_ls, W_k, b_k, W_q, b_q, W_v, b_v, W_p1, b_p1,
           g_p1, be_p1, W_p2, b_p2, W_a1, b_a1, g_a1, be_a1, W_a2, b_a2,
           W_le, b_le):
    pos_t = jnp.swapaxes(pos, 1, 2)
    r2 = lambda a: a.reshape(1, -1)
    grid = (B, N // T)

    p1 = pl.pallas_call(
        _p1_body,
        grid=grid,
        in_specs=[
            pl.BlockSpec((1, T, IN_CH), lambda b, t: (b, t, 0)),
            pl.BlockSpec((1, N, 3), lambda b, t: (b, 0, 0)),
            pl.BlockSpec((1, 3, N), lambda b, t: (b, 0, 0)),
            _full((IN_CH, DIM)), _full((1, DIM)),
            _full((DIM, DIM)), _full((1, DIM)),
            _full((DIM, DIM)), _full((1, DIM)),
            _full((DIM, DIM)), _full((1, DIM)),
        ],
        out_specs=[
            pl.BlockSpec((1, T, IN_CH), lambda b, t: (b, t, 0)),
            pl.BlockSpec((1, T, DIM), lambda b, t: (b, t, 0)),
            pl.BlockSpec((1, T, DIM), lambda b, t: (b, t, 0)),
            pl.BlockSpec((1, T, K), lambda b, t: (b, t, 0)),
            pl.BlockSpec((8, 128), lambda b, t: (0, 0)),
        ],
        out_shape=[
            jax.ShapeDtypeStruct((B, N, IN_CH), F32),
            jax.ShapeDtypeStruct((B, N, DIM), F32),
            jax.ShapeDtypeStruct((B, N, DIM), F32),
            jax.ShapeDtypeStruct((B, N, K), jnp.int32),
            jax.ShapeDtypeStruct((8, 128), F32),
        ],
    )
    keypos, value, query, idxg, s1 = p1(
        x, pos, pos_t, W_ls, r2(b_ls), W_k, r2(b_k), W_q, r2(b_q),
        W_v, r2(b_v))

    ar = jnp.arange(M, dtype=jnp.int32)
    oidx3 = ((ar % K) * (B * N) + ar // K).reshape(NW, NCHUNK, CH)
    idx3 = idxg.reshape(NW, NCHUNK, CH)
    keypos_flat = keypos.reshape(B * N, IN_CH)
    kpg_flat = _sc_gather(keypos_flat, idx3, oidx3)
    kpg = kpg_flat.reshape(K, B, N, IN_CH)

    p3 = pl.pallas_call(
        _p3_body,
        grid=grid,
        in_specs=[
            pl.BlockSpec((K, 1, T, IN_CH), lambda b, t: (0, b, t, 0)),
            pl.BlockSpec((1, T, 3), lambda b, t: (b, t, 0)),
            pl.BlockSpec((1, T, DIM), lambda b, t: (b, t, 0)),
            pl.BlockSpec((1, T, DIM), lambda b, t: (b, t, 0)),
            _full((8, 128)),
            _full((3, POS_H)), _full((1, POS_H)),
            _full((1, POS_H)), _full((1, POS_H)),
            _full((POS_H, DIM)), _full((1, DIM)),
            _full((DIM, H)), _full((1, H)),
        ],
        out_specs=[
            pl.BlockSpec((K, 1, T, DIM), lambda b, t: (0, b, t, 0)),
            pl.BlockSpec((K, 1, T, DIM), lambda b, t: (0, b, t, 0)),
            pl.BlockSpec((8, H), lambda b, t: (0, 0)),
        ],
        out_shape=[
            jax.ShapeDtypeStruct((K, B, N, DIM), F32),
            jax.ShapeDtypeStruct((K, B, N, DIM), F32),
            jax.ShapeDtypeStruct((8, H), F32),
        ],
    )
    u, v, s2 = p3(kpg, pos, query, value, s1,
                  W_p1, r2(b_p1), r2(g_p1), r2(be_p1), W_p2, r2(b_p2),
                  W_a1, r2(b_a1))

    p4 = pl.pallas_call(
        _p4_body,
        grid=grid,
        in_specs=[
            pl.BlockSpec((K, 1, T, DIM), lambda b, t: (0, b, t, 0)),
            pl.BlockSpec((K, 1, T, DIM), lambda b, t: (0, b, t, 0)),
            pl.BlockSpec((1, T, IN_CH), lambda b, t: (b, t, 0)),
            _full((8, H)),
            _full((DIM, H)), _full((1, H)),
            _full((1, H)), _full((1, H)),
            _full((H, DIM)), _full((1, DIM)),
            _full((DIM, IN_CH)), _full((1, IN_CH)),
        ],
        out_specs=pl.BlockSpec((1, T, IN_CH), lambda b, t: (b, t, 0)),
        out_shape=jax.ShapeDtypeStruct((B, N, IN_CH), F32),
    )
    y = p4(u, v, x, s2, W_a1, r2(b_a1), r2(g_a1), r2(be_a1),
           W_a2, r2(b_a2), W_le, r2(b_le))
    return y


# speedup vs baseline: 1.4785x; 1.1946x over previous
import functools

import jax
import jax.numpy as jnp
from jax import lax
from jax.experimental import pallas as pl
from jax.experimental.pallas import tpu as pltpu
from jax.experimental.pallas import tpu_sc as plsc

B, N, IN_CH, DIM, K, POS_H, MULT = 4, 2048, 128, 64, 16, 64, 4
H = DIM * MULT
T = 512
M = B * N * K
EPS = 1e-3
F32 = jnp.float32
IMAX = 0x7FFFFFFF
NW = 32
CH = 128
NCHUNK = M // NW // CH


def _p1_body(x_ref, pos_ref, post_ref, wls_ref, bls_ref, wk_ref, bk_ref,
             wq_ref, bq_ref, wv_ref, bv_ref,
             keypos_ref, value_ref, query_ref, idx_ref, s1_ref):
    b = pl.program_id(0)
    t = pl.program_id(1)

    pos_t = post_ref[0]
    row0 = pl.multiple_of(t * T, T)
    pos_tile = pos_ref[0, pl.ds(row0, T), :]

    x = x_ref[0]
    h = jnp.dot(x, wls_ref[...], preferred_element_type=F32) + bls_ref[...]
    keyf = jnp.dot(h, wk_ref[...], preferred_element_type=F32) + bk_ref[...]
    keypos_ref[0] = jnp.concatenate(
        [keyf, pos_tile, jnp.zeros((T, IN_CH - DIM - 3), F32)], axis=1)
    value_ref[0] = jnp.dot(h, wq_ref[...], preferred_element_type=F32) + bq_ref[...]
    query_ref[0] = jnp.dot(h, wv_ref[...], preferred_element_type=F32) + bv_ref[...]

    sq_tile = jnp.sum(pos_tile * pos_tile, axis=1, keepdims=True)
    sq_full = jnp.sum(pos_t * pos_t, axis=0, keepdims=True)
    e = jnp.dot(pos_tile, pos_t, preferred_element_type=F32)
    dist = (sq_tile - 2.0 * e) + sq_full

    bits = dist.view(jnp.int32)
    bits = bits ^ ((bits >> 31) & jnp.int32(0x7FFFFFFF))
    iota = jax.lax.broadcasted_iota(jnp.int32, (T, N), 1)
    packed = (bits & jnp.int32(~2047)) | iota

    base = b * N
    idx_cols = []
    for k in range(K):
        mn = jnp.min(packed, axis=1, keepdims=True)
        hit = packed == mn
        packed = jnp.where(hit, jnp.int32(IMAX), packed)
        idx_cols.append((mn & 2047) + base)
    idx_ref[0] = jnp.concatenate(idx_cols, axis=1)

    mh = (packed == jnp.int32(IMAX)).astype(F32)
    pf = pos_ref[0, :, :]
    p0 = pf[:, 0:1]
    p1 = pf[:, 1:2]
    p2 = pf[:, 2:3]
    cat = jnp.concatenate(
        [pf, p0 * p0, p0 * p1, p0 * p2, p1 * p1, p1 * p2, p2 * p2], axis=1)
    sq12 = jnp.dot(mh, cat, preferred_element_type=F32)
    s = sq12[:, 0:3]
    q = sq12[:, 3:9]
    a0 = pos_tile[:, 0:1]
    a1 = pos_tile[:, 1:2]
    a2 = pos_tile[:, 2:3]
    s0 = s[:, 0:1]
    s1c = s[:, 1:2]
    s2c = s[:, 2:3]
    kf = jnp.float32(K)
    c_tile = jnp.concatenate([
        kf * pos_tile - s,
        kf * a0 * a0 - 2.0 * a0 * s0 + q[:, 0:1],
        kf * a0 * a1 - a0 * s1c - a1 * s0 + q[:, 1:2],
        kf * a0 * a2 - a0 * s2c - a2 * s0 + q[:, 2:3],
        kf * a1 * a1 - 2.0 * a1 * s1c + q[:, 3:4],
        kf * a1 * a2 - a1 * s2c - a2 * s1c + q[:, 4:5],
        kf * a2 * a2 - 2.0 * a2 * s2c + q[:, 5:6],
    ], axis=1)
    sums = jnp.sum(c_tile, axis=0, keepdims=True)
    upd = jnp.concatenate(
        [jnp.concatenate([sums, jnp.zeros((1, 128 - 9), F32)], axis=1),
         jnp.zeros((7, 128), F32)], axis=0)

    @pl.when((b == 0) & (t == 0))
    def _():
        s1_ref[...] = jnp.zeros_like(s1_ref)

    s1_ref[...] += upd


def _sc_gather_body(keytab, idx_hbm, oidx_hbm, keyg,
                    idx_v, oidx_v, rows_a, rows_b, sem_a, sem_b):
    wid = lax.axis_index("s") * 2 + lax.axis_index("c")
    pltpu.sync_copy(idx_hbm.at[wid], idx_v)
    pltpu.sync_copy(oidx_hbm.at[wid], oidx_v)

    def chunk(j, carry):
        a = pltpu.async_copy(keytab.at[idx_v.at[2 * j]], rows_a, sem_a)
        b2 = pltpu.async_copy(keytab.at[idx_v.at[2 * j + 1]], rows_b, sem_b)
        a.wait()
        c = pltpu.async_copy(rows_a, keyg.at[oidx_v.at[2 * j]], sem_a)
        b2.wait()
        d = pltpu.async_copy(rows_b, keyg.at[oidx_v.at[2 * j + 1]], sem_b)
        c.wait()
        d.wait()
        return carry

    lax.fori_loop(0, NCHUNK // 2, chunk, 0)


def _bn1_consts(s1_ref, wp1_ref, bp1_ref, gp1_ref, bep1_ref):
    sums = s1_ref[0:1, :]
    s3 = sums[:, 0:3] / M
    c6 = sums[:, 3:9] / M
    m0 = s3[:, 0:1]
    m1 = s3[:, 1:2]
    m2 = s3[:, 2:3]
    r0 = jnp.concatenate([c6[:, 0:1] - m0 * m0, c6[:, 1:2] - m0 * m1,
                          c6[:, 2:3] - m0 * m2], axis=1)
    r1 = jnp.concatenate([c6[:, 1:2] - m0 * m1, c6[:, 3:4] - m1 * m1,
                          c6[:, 4:5] - m1 * m2], axis=1)
    r2 = jnp.concatenate([c6[:, 2:3] - m0 * m2, c6[:, 4:5] - m1 * m2,
                          c6[:, 5:6] - m2 * m2], axis=1)
    cov = jnp.concatenate([r0, r1, r2], axis=0)
    wp1 = wp1_ref[...]
    cw = jnp.dot(cov, wp1, preferred_element_type=F32)
    var1 = jnp.sum(wp1 * cw, axis=0, keepdims=True)
    mean1 = jnp.dot(s3, wp1, preferred_element_type=F32) + bp1_ref[...]
    scale1 = gp1_ref[...] * jax.lax.rsqrt(var1 + EPS)
    shift1 = bep1_ref[...] - mean1 * scale1
    return scale1, shift1


def _p3_body(keyg_ref, pos_ref, query_ref, value_ref, s1_ref,
             wp1_ref, bp1_ref, gp1_ref, bep1_ref, wp2_ref, bp2_ref,
             wa1_ref, ba1_ref,
             u_ref, v_ref, s2_ref):
    b = pl.program_id(0)
    t = pl.program_id(1)

    scale1, shift1 = _bn1_consts(s1_ref, wp1_ref, bp1_ref, gp1_ref, bep1_ref)
    pos_tile = pos_ref[0]
    query = query_ref[0]
    value = value_ref[0]

    s_acc = jnp.zeros((1, H), F32)
    q_acc = jnp.zeros((1, H), F32)
    for k in range(K):
        kgrow = keyg_ref[k, 0]
        prel = pos_tile - kgrow[:, DIM:DIM + 3]
        p = jnp.dot(prel, wp1_ref[...], preferred_element_type=F32) + bp1_ref[...]
        pe = jax.nn.relu(p * scale1 + shift1)
        pe = jnp.dot(pe, wp2_ref[...], preferred_element_type=F32) + bp2_ref[...]
        u = (query - kgrow[:, :DIM]) + pe
        u_ref[k, 0] = u
        v_ref[k, 0] = value + pe
        a = jnp.dot(u, wa1_ref[...], preferred_element_type=F32) + ba1_ref[...]
        s_acc = s_acc + jnp.sum(a, axis=0, keepdims=True)
        q_acc = q_acc + jnp.sum(a * a, axis=0, keepdims=True)

    @pl.when((b == 0) & (t == 0))
    def _():
        s2_ref[...] = jnp.zeros_like(s2_ref)

    s2_ref[...] += jnp.concatenate(
        [s_acc, q_acc, jnp.zeros((6, H), F32)], axis=0)


def _p4_body(u_ref, v_ref, x_ref, s2_ref,
             wa1_ref, ba1_ref, ga1_ref, bea1_ref, wa2_ref, ba2_ref,
             wle_ref, ble_ref, y_ref):
    s2 = s2_ref[...]
    mean2 = s2[0:1, :] / M
    var2 = s2[1:2, :] / M - mean2 * mean2
    scale2 = ga1_ref[...] * jax.lax.rsqrt(var2 + EPS)
    shift2 = bea1_ref[...] - mean2 * scale2

    acc = jnp.zeros((T, DIM), F32)
    for k in range(K):
        u = u_ref[k, 0]
        a = jnp.dot(u, wa1_ref[...], preferred_element_type=F32) + ba1_ref[...]
        a = jax.nn.relu(a * scale2 + shift2)
        logit = jnp.dot(a, wa2_ref[...], preferred_element_type=F32) + ba2_ref[...]
        mx = jnp.max(logit, axis=1, keepdims=True)
        ex = jnp.exp(logit - mx)
        p = ex / jnp.sum(ex, axis=1, keepdims=True)
        acc = acc + p * v_ref[k, 0]
    y = jnp.dot(acc, wle_ref[...], preferred_element_type=F32) + ble_ref[...]
    y_ref[0] = y + x_ref[0]


def _full(shape):
    nd = len(shape)
    return pl.BlockSpec(shape, lambda b, t, _nd=nd: (0,) * _nd)


def _sc_gather(keypos_flat, idx3, oidx3):
    mesh = plsc.VectorSubcoreMesh(core_axis_name="c", subcore_axis_name="s")
    run = functools.partial(
        pl.kernel,
        out_type=jax.ShapeDtypeStruct((M, IN_CH), F32),
        mesh=mesh,
        scratch_types=[
            pltpu.VMEM((NCHUNK, CH), jnp.int32),
            pltpu.VMEM((NCHUNK, CH), jnp.int32),
            pltpu.VMEM((CH, IN_CH), F32),
            pltpu.VMEM((CH, IN_CH), F32),
            pltpu.SemaphoreType.DMA,
            pltpu.SemaphoreType.DMA,
        ],
    )(_sc_gather_body)
    return run(keypos_flat, idx3, oidx3)


def kernel(x, pos, W_ls, b_ls, W_k, b_k, W_q, b_q, W_v, b_v, W_p1, b_p1,
           g_p1, be_p1, W_p2, b_p2, W_a1, b_a1, g_a1, be_a1, W_a2, b_a2,
           W_le, b_le):
    pos_t = jnp.swapaxes(pos, 1, 2)
    r2 = lambda a: a.reshape(1, -1)
    grid = (B, N // T)

    p1 = pl.pallas_call(
        _p1_body,
        grid=grid,
        in_specs=[
            pl.BlockSpec((1, T, IN_CH), lambda b, t: (b, t, 0)),
            pl.BlockSpec((1, N, 3), lambda b, t: (b, 0, 0)),
            pl.BlockSpec((1, 3, N), lambda b, t: (b, 0, 0)),
            _full((IN_CH, DIM)), _full((1, DIM)),
            _full((DIM, DIM)), _full((1, DIM)),
            _full((DIM, DIM)), _full((1, DIM)),
            _full((DIM, DIM)), _full((1, DIM)),
        ],
        out_specs=[
            pl.BlockSpec((1, T, IN_CH), lambda b, t: (b, t, 0)),
            pl.BlockSpec((1, T, DIM), lambda b, t: (b, t, 0)),
            pl.BlockSpec((1, T, DIM), lambda b, t: (b, t, 0)),
            pl.BlockSpec((1, T, K), lambda b, t: (b, t, 0)),
            pl.BlockSpec((8, 128), lambda b, t: (0, 0)),
        ],
        out_shape=[
            jax.ShapeDtypeStruct((B, N, IN_CH), F32),
            jax.ShapeDtypeStruct((B, N, DIM), F32),
            jax.ShapeDtypeStruct((B, N, DIM), F32),
            jax.ShapeDtypeStruct((B, N, K), jnp.int32),
            jax.ShapeDtypeStruct((8, 128), F32),
        ],
    )
    keypos, value, query, idxg, s1 = p1(
        x, pos, pos_t, W_ls, r2(b_ls), W_k, r2(b_k), W_q, r2(b_q),
        W_v, r2(b_v))

    ar = jnp.arange(M, dtype=jnp.int32)
    oidx3 = ((ar % K) * (B * N) + ar // K).reshape(NW, NCHUNK, CH)
    idx3 = idxg.reshape(NW, NCHUNK, CH)
    keypos_flat = keypos.reshape(B * N, IN_CH)
    kpg_flat = _sc_gather(keypos_flat, idx3, oidx3)
    kpg = kpg_flat.reshape(K, B, N, IN_CH)

    p3 = pl.pallas_call(
        _p3_body,
        grid=grid,
        in_specs=[
            pl.BlockSpec((K, 1, T, IN_CH), lambda b, t: (0, b, t, 0)),
            pl.BlockSpec((1, T, 3), lambda b, t: (b, t, 0)),
            pl.BlockSpec((1, T, DIM), lambda b, t: (b, t, 0)),
            pl.BlockSpec((1, T, DIM), lambda b, t: (b, t, 0)),
            _full((8, 128)),
            _full((3, POS_H)), _full((1, POS_H)),
            _full((1, POS_H)), _full((1, POS_H)),
            _full((POS_H, DIM)), _full((1, DIM)),
            _full((DIM, H)), _full((1, H)),
        ],
        out_specs=[
            pl.BlockSpec((K, 1, T, DIM), lambda b, t: (0, b, t, 0)),
            pl.BlockSpec((K, 1, T, DIM), lambda b, t: (0, b, t, 0)),
            pl.BlockSpec((8, H), lambda b, t: (0, 0)),
        ],
        out_shape=[
            jax.ShapeDtypeStruct((K, B, N, DIM), F32),
            jax.ShapeDtypeStruct((K, B, N, DIM), F32),
            jax.ShapeDtypeStruct((8, H), F32),
        ],
    )
    u, v, s2 = p3(kpg, pos, query, value, s1,
                  W_p1, r2(b_p1), r2(g_p1), r2(be_p1), W_p2, r2(b_p2),
                  W_a1, r2(b_a1))

    p4 = pl.pallas_call(
        _p4_body,
        grid=grid,
        in_specs=[
            pl.BlockSpec((K, 1, T, DIM), lambda b, t: (0, b, t, 0)),
            pl.BlockSpec((K, 1, T, DIM), lambda b, t: (0, b, t, 0)),
            pl.BlockSpec((1, T, IN_CH), lambda b, t: (b, t, 0)),
            _full((8, H)),
            _full((DIM, H)), _full((1, H)),
            _full((1, H)), _full((1, H)),
            _full((H, DIM)), _full((1, DIM)),
            _full((DIM, IN_CH)), _full((1, IN_CH)),
        ],
        out_specs=pl.BlockSpec((1, T, IN_CH), lambda b, t: (b, t, 0)),
        out_shape=jax.ShapeDtypeStruct((B, N, IN_CH), F32),
    )
    y = p4(u, v, x, s2, W_a1, r2(b_a1), r2(g_a1), r2(be_a1),
           W_a2, r2(b_a2), W_le, r2(b_le))
    return y


# speedup vs baseline: 1.4930x; 1.0098x over previous
import functools

import jax
import jax.numpy as jnp
from jax import lax
from jax.experimental import pallas as pl
from jax.experimental.pallas import tpu as pltpu
from jax.experimental.pallas import tpu_sc as plsc

B, N, IN_CH, DIM, K, POS_H, MULT = 4, 2048, 128, 64, 16, 64, 4
H = DIM * MULT
T = 1024
M = B * N * K
EPS = 1e-3
F32 = jnp.float32
IMAX = 0x7FFFFFFF
NW = 32
CH = 128
NCHUNK = M // NW // CH


def _p1_body(x_ref, pos_ref, post_ref, wls_ref, bls_ref, wk_ref, bk_ref,
             wq_ref, bq_ref, wv_ref, bv_ref,
             keypos_ref, value_ref, query_ref, idx_ref, s1_ref):
    b = pl.program_id(0)
    t = pl.program_id(1)

    pos_t = post_ref[0]
    row0 = pl.multiple_of(t * T, T)
    pos_tile = pos_ref[0, pl.ds(row0, T), :]

    x = x_ref[0]
    h = jnp.dot(x, wls_ref[...], preferred_element_type=F32) + bls_ref[...]
    keyf = jnp.dot(h, wk_ref[...], preferred_element_type=F32) + bk_ref[...]
    keypos_ref[0] = jnp.concatenate(
        [keyf, pos_tile, jnp.zeros((T, IN_CH - DIM - 3), F32)], axis=1)
    value_ref[0] = jnp.dot(h, wq_ref[...], preferred_element_type=F32) + bq_ref[...]
    query_ref[0] = jnp.dot(h, wv_ref[...], preferred_element_type=F32) + bv_ref[...]

    sq_tile = jnp.sum(pos_tile * pos_tile, axis=1, keepdims=True)
    sq_full = jnp.sum(pos_t * pos_t, axis=0, keepdims=True)
    e = jnp.dot(pos_tile, pos_t, preferred_element_type=F32)
    dist = (sq_tile - 2.0 * e) + sq_full

    bits = dist.view(jnp.int32)
    bits = bits ^ ((bits >> 31) & jnp.int32(0x7FFFFFFF))
    iota = jax.lax.broadcasted_iota(jnp.int32, (T, N), 1)
    packed = (bits & jnp.int32(~2047)) | iota

    base = b * N
    idx_cols = []
    for k in range(K):
        mn = jnp.min(packed, axis=1, keepdims=True)
        hit = packed == mn
        packed = jnp.where(hit, jnp.int32(IMAX), packed)
        idx_cols.append((mn & 2047) + base)
    idx_ref[0] = jnp.concatenate(idx_cols, axis=1)

    mh = (packed == jnp.int32(IMAX)).astype(F32)
    pf = pos_ref[0, :, :]
    p0 = pf[:, 0:1]
    p1 = pf[:, 1:2]
    p2 = pf[:, 2:3]
    cat = jnp.concatenate(
        [pf, p0 * p0, p0 * p1, p0 * p2, p1 * p1, p1 * p2, p2 * p2], axis=1)
    sq12 = jnp.dot(mh, cat, preferred_element_type=F32)
    s = sq12[:, 0:3]
    q = sq12[:, 3:9]
    a0 = pos_tile[:, 0:1]
    a1 = pos_tile[:, 1:2]
    a2 = pos_tile[:, 2:3]
    s0 = s[:, 0:1]
    s1c = s[:, 1:2]
    s2c = s[:, 2:3]
    kf = jnp.float32(K)
    c_tile = jnp.concatenate([
        kf * pos_tile - s,
        kf * a0 * a0 - 2.0 * a0 * s0 + q[:, 0:1],
        kf * a0 * a1 - a0 * s1c - a1 * s0 + q[:, 1:2],
        kf * a0 * a2 - a0 * s2c - a2 * s0 + q[:, 2:3],
        kf * a1 * a1 - 2.0 * a1 * s1c + q[:, 3:4],
        kf * a1 * a2 - a1 * s2c - a2 * s1c + q[:, 4:5],
        kf * a2 * a2 - 2.0 * a2 * s2c + q[:, 5:6],
    ], axis=1)
    sums = jnp.sum(c_tile, axis=0, keepdims=True)
    upd = jnp.concatenate(
        [jnp.concatenate([sums, jnp.zeros((1, 128 - 9), F32)], axis=1),
         jnp.zeros((7, 128), F32)], axis=0)

    @pl.when((b == 0) & (t == 0))
    def _():
        s1_ref[...] = jnp.zeros_like(s1_ref)

    s1_ref[...] += upd


def _sc_gather_body(keytab, idx_hbm, oidx_hbm, keyg,
                    idx_v, oidx_v, rows_a, rows_b, sem_a, sem_b):
    wid = lax.axis_index("s") * 2 + lax.axis_index("c")
    pltpu.sync_copy(idx_hbm.at[wid], idx_v)
    pltpu.sync_copy(oidx_hbm.at[wid], oidx_v)

    def chunk(j, carry):
        a = pltpu.async_copy(keytab.at[idx_v.at[2 * j]], rows_a, sem_a)
        b2 = pltpu.async_copy(keytab.at[idx_v.at[2 * j + 1]], rows_b, sem_b)
        a.wait()
        c = pltpu.async_copy(rows_a, keyg.at[oidx_v.at[2 * j]], sem_a)
        b2.wait()
        d = pltpu.async_copy(rows_b, keyg.at[oidx_v.at[2 * j + 1]], sem_b)
        c.wait()
        d.wait()
        return carry

    lax.fori_loop(0, NCHUNK // 2, chunk, 0)


def _bn1_consts(s1_ref, wp1_ref, bp1_ref, gp1_ref, bep1_ref):
    sums = s1_ref[0:1, :]
    s3 = sums[:, 0:3] / M
    c6 = sums[:, 3:9] / M
    m0 = s3[:, 0:1]
    m1 = s3[:, 1:2]
    m2 = s3[:, 2:3]
    r0 = jnp.concatenate([c6[:, 0:1] - m0 * m0, c6[:, 1:2] - m0 * m1,
                          c6[:, 2:3] - m0 * m2], axis=1)
    r1 = jnp.concatenate([c6[:, 1:2] - m0 * m1, c6[:, 3:4] - m1 * m1,
                          c6[:, 4:5] - m1 * m2], axis=1)
    r2 = jnp.concatenate([c6[:, 2:3] - m0 * m2, c6[:, 4:5] - m1 * m2,
                          c6[:, 5:6] - m2 * m2], axis=1)
    cov = jnp.concatenate([r0, r1, r2], axis=0)
    wp1 = wp1_ref[...]
    cw = jnp.dot(cov, wp1, preferred_element_type=F32)
    var1 = jnp.sum(wp1 * cw, axis=0, keepdims=True)
    mean1 = jnp.dot(s3, wp1, preferred_element_type=F32) + bp1_ref[...]
    scale1 = gp1_ref[...] * jax.lax.rsqrt(var1 + EPS)
    shift1 = bep1_ref[...] - mean1 * scale1
    return scale1, shift1


def _p3_body(keyg_ref, pos_ref, query_ref, value_ref, s1_ref,
             wp1_ref, bp1_ref, gp1_ref, bep1_ref, wp2_ref, bp2_ref,
             wa1_ref, ba1_ref,
             u_ref, v_ref, s2_ref):
    b = pl.program_id(0)
    t = pl.program_id(1)

    scale1, shift1 = _bn1_consts(s1_ref, wp1_ref, bp1_ref, gp1_ref, bep1_ref)
    pos_tile = pos_ref[0]
    query = query_ref[0]
    value = value_ref[0]

    s_acc = jnp.zeros((1, H), F32)
    q_acc = jnp.zeros((1, H), F32)
    for k in range(K):
        kgrow = keyg_ref[k, 0]
        prel = pos_tile - kgrow[:, DIM:DIM + 3]
        p = jnp.dot(prel, wp1_ref[...], preferred_element_type=F32) + bp1_ref[...]
        pe = jax.nn.relu(p * scale1 + shift1)
        pe = jnp.dot(pe, wp2_ref[...], preferred_element_type=F32) + bp2_ref[...]
        u = (query - kgrow[:, :DIM]) + pe
        u_ref[k, 0] = u
        v_ref[k, 0] = value + pe
        a = jnp.dot(u, wa1_ref[...], preferred_element_type=F32) + ba1_ref[...]
        s_acc = s_acc + jnp.sum(a, axis=0, keepdims=True)
        q_acc = q_acc + jnp.sum(a * a, axis=0, keepdims=True)

    @pl.when((b == 0) & (t == 0))
    def _():
        s2_ref[...] = jnp.zeros_like(s2_ref)

    s2_ref[...] += jnp.concatenate(
        [s_acc, q_acc, jnp.zeros((6, H), F32)], axis=0)


def _p4_body(u_ref, v_ref, x_ref, s2_ref,
             wa1_ref, ba1_ref, ga1_ref, bea1_ref, wa2_ref, ba2_ref,
             wle_ref, ble_ref, y_ref):
    s2 = s2_ref[...]
    mean2 = s2[0:1, :] / M
    var2 = s2[1:2, :] / M - mean2 * mean2
    scale2 = ga1_ref[...] * jax.lax.rsqrt(var2 + EPS)
    shift2 = bea1_ref[...] - mean2 * scale2

    acc = jnp.zeros((T, DIM), F32)
    for k in range(K):
        u = u_ref[k, 0]
        a = jnp.dot(u, wa1_ref[...], preferred_element_type=F32) + ba1_ref[...]
        a = jax.nn.relu(a * scale2 + shift2)
        logit = jnp.dot(a, wa2_ref[...], preferred_element_type=F32) + ba2_ref[...]
        mx = jnp.max(logit, axis=1, keepdims=True)
        ex = jnp.exp(logit - mx)
        p = ex / jnp.sum(ex, axis=1, keepdims=True)
        acc = acc + p * v_ref[k, 0]
    y = jnp.dot(acc, wle_ref[...], preferred_element_type=F32) + ble_ref[...]
    y_ref[0] = y + x_ref[0]


def _full(shape):
    nd = len(shape)
    return pl.BlockSpec(shape, lambda b, t, _nd=nd: (0,) * _nd)


def _sc_gather(keypos_flat, idx3, oidx3):
    mesh = plsc.VectorSubcoreMesh(core_axis_name="c", subcore_axis_name="s")
    run = functools.partial(
        pl.kernel,
        out_type=jax.ShapeDtypeStruct((M, IN_CH), F32),
        mesh=mesh,
        scratch_types=[
            pltpu.VMEM((NCHUNK, CH), jnp.int32),
            pltpu.VMEM((NCHUNK, CH), jnp.int32),
            pltpu.VMEM((CH, IN_CH), F32),
            pltpu.VMEM((CH, IN_CH), F32),
            pltpu.SemaphoreType.DMA,
            pltpu.SemaphoreType.DMA,
        ],
    )(_sc_gather_body)
    return run(keypos_flat, idx3, oidx3)


def kernel(x, pos, W_ls, b_ls, W_k, b_k, W_q, b_q, W_v, b_v, W_p1, b_p1,
           g_p1, be_p1, W_p2, b_p2, W_a1, b_a1, g_a1, be_a1, W_a2, b_a2,
           W_le, b_le):
    pos_t = jnp.swapaxes(pos, 1, 2)
    r2 = lambda a: a.reshape(1, -1)
    grid = (B, N // T)

    p1 = pl.pallas_call(
        _p1_body,
        grid=grid,
        in_specs=[
            pl.BlockSpec((1, T, IN_CH), lambda b, t: (b, t, 0)),
            pl.BlockSpec((1, N, 3), lambda b, t: (b, 0, 0)),
            pl.BlockSpec((1, 3, N), lambda b, t: (b, 0, 0)),
            _full((IN_CH, DIM)), _full((1, DIM)),
            _full((DIM, DIM)), _full((1, DIM)),
            _full((DIM, DIM)), _full((1, DIM)),
            _full((DIM, DIM)), _full((1, DIM)),
        ],
        out_specs=[
            pl.BlockSpec((1, T, IN_CH), lambda b, t: (b, t, 0)),
            pl.BlockSpec((1, T, DIM), lambda b, t: (b, t, 0)),
            pl.BlockSpec((1, T, DIM), lambda b, t: (b, t, 0)),
            pl.BlockSpec((1, T, K), lambda b, t: (b, t, 0)),
            pl.BlockSpec((8, 128), lambda b, t: (0, 0)),
        ],
        out_shape=[
            jax.ShapeDtypeStruct((B, N, IN_CH), F32),
            jax.ShapeDtypeStruct((B, N, DIM), F32),
            jax.ShapeDtypeStruct((B, N, DIM), F32),
            jax.ShapeDtypeStruct((B, N, K), jnp.int32),
            jax.ShapeDtypeStruct((8, 128), F32),
        ],
    )
    keypos, value, query, idxg, s1 = p1(
        x, pos, pos_t, W_ls, r2(b_ls), W_k, r2(b_k), W_q, r2(b_q),
        W_v, r2(b_v))

    ar = jnp.arange(M, dtype=jnp.int32)
    oidx3 = ((ar % K) * (B * N) + ar // K).reshape(NW, NCHUNK, CH)
    idx3 = idxg.reshape(NW, NCHUNK, CH)
    keypos_flat = keypos.reshape(B * N, IN_CH)
    kpg_flat = _sc_gather(keypos_flat, idx3, oidx3)
    kpg = kpg_flat.reshape(K, B, N, IN_CH)

    p3 = pl.pallas_call(
        _p3_body,
        grid=grid,
        in_specs=[
            pl.BlockSpec((K, 1, T, IN_CH), lambda b, t: (0, b, t, 0)),
            pl.BlockSpec((1, T, 3), lambda b, t: (b, t, 0)),
            pl.BlockSpec((1, T, DIM), lambda b, t: (b, t, 0)),
            pl.BlockSpec((1, T, DIM), lambda b, t: (b, t, 0)),
            _full((8, 128)),
            _full((3, POS_H)), _full((1, POS_H)),
            _full((1, POS_H)), _full((1, POS_H)),
            _full((POS_H, DIM)), _full((1, DIM)),
            _full((DIM, H)), _full((1, H)),
        ],
        out_specs=[
            pl.BlockSpec((K, 1, T, DIM), lambda b, t: (0, b, t, 0)),
            pl.BlockSpec((K, 1, T, DIM), lambda b, t: (0, b, t, 0)),
            pl.BlockSpec((8, H), lambda b, t: (0, 0)),
        ],
        out_shape=[
            jax.ShapeDtypeStruct((K, B, N, DIM), F32),
            jax.ShapeDtypeStruct((K, B, N, DIM), F32),
            jax.ShapeDtypeStruct((8, H), F32),
        ],
    )
    u, v, s2 = p3(kpg, pos, query, value, s1,
                  W_p1, r2(b_p1), r2(g_p1), r2(be_p1), W_p2, r2(b_p2),
                  W_a1, r2(b_a1))

    p4 = pl.pallas_call(
        _p4_body,
        grid=grid,
        in_specs=[
            pl.BlockSpec((K, 1, T, DIM), lambda b, t: (0, b, t, 0)),
            pl.BlockSpec((K, 1, T, DIM), lambda b, t: (0, b, t, 0)),
            pl.BlockSpec((1, T, IN_CH), lambda b, t: (b, t, 0)),
            _full((8, H)),
            _full((DIM, H)), _full((1, H)),
            _full((1, H)), _full((1, H)),
            _full((H, DIM)), _full((1, DIM)),
            _full((DIM, IN_CH)), _full((1, IN_CH)),
        ],
        out_specs=pl.BlockSpec((1, T, IN_CH), lambda b, t: (b, t, 0)),
        out_shape=jax.ShapeDtypeStruct((B, N, IN_CH), F32),
    )
    y = p4(u, v, x, s2, W_a1, r2(b_a1), r2(g_a1), r2(be_a1),
           W_a2, r2(b_a2), W_le, r2(b_le))
    return y
